# Initial kernel scaffold; baseline (speedup 1.0000x reference)
import jax
import jax.numpy as jnp
from jax.experimental import pallas as pl


def kernel(x, edge_index, edge_attr, batch, random_feats, params):
    raise NotImplementedError("write your pallas kernel here")


# speedup vs baseline: 1.6396x; 1.6396x over previous
import functools

import jax
import jax.numpy as jnp
from jax import lax
from jax.experimental import pallas as pl
from jax.experimental.pallas import tpu as pltpu
from jax.experimental.pallas import tpu_sc as plsc

N = 10000
E = 320000
DIN = 144
DH = 80
H = 128
NG = 64
NCLS = 10
LANES = 16

NC = 2
NS = 16
NW = NC * NS
EPW = E // NW
CHUNK = 80
NCHUNK = EPW // CHUNK
ZB = 200
NBLK = N // ZB
NBLK_IT = -(-NBLK // NS)

RB = 2000
NRB = N // RB
EB = 8000


def _sc_agg_body(xa_hbm, xb_hbm, ea_hbm, eb_hbm, src_hbm, dst_hbm, out_hbm,
                 src_v, dst_v, x_v, e_v, z_v, acc_sh, sem):
    c = lax.axis_index("c")
    s = lax.axis_index("s")
    wid = c * NS + s

    zv = jnp.zeros((LANES,), jnp.float32)

    def zrow(r, carry):
        for j in range(DH // LANES):
            z_v[r, pl.ds(j * LANES, LANES)] = zv
        return carry

    lax.fori_loop(0, ZB, zrow, 0)

    for p, (x_hbm, e_hbm) in enumerate(((xa_hbm, ea_hbm), (xb_hbm, eb_hbm))):
        def zcp(k, carry):
            b = s + NS * k

            @pl.when(b < NBLK)
            def _():
                pltpu.sync_copy(z_v, acc_sh.at[pl.ds(b * ZB, ZB), :])

            return carry

        lax.fori_loop(0, NBLK_IT, zcp, 0)
        plsc.subcore_barrier()

        def chunk_body(i, carry):
            base = wid * EPW + i * CHUNK
            pltpu.sync_copy(src_hbm.at[pl.ds(base, CHUNK)], src_v)
            pltpu.sync_copy(dst_hbm.at[pl.ds(base, CHUNK)], dst_v)
            pltpu.async_copy(x_hbm.at[src_v], x_v, sem).wait()
            pltpu.sync_copy(e_hbm.at[pl.ds(base, CHUNK), :], e_v)

            def row(r, inner):
                for j in range(DH // LANES):
                    sl = pl.ds(j * LANES, LANES)
                    x_v[r, sl] = jnp.maximum(x_v[r, sl] + e_v[r, sl], 0.0)
                return inner

            lax.fori_loop(0, CHUNK, row, 0)
            pltpu.sync_copy(x_v, acc_sh.at[dst_v], add=True)
            return carry

        lax.fori_loop(0, NCHUNK, chunk_body, 0)
        plsc.subcore_barrier()

        def ocp(k, carry):
            b = s + NS * k

            @pl.when(b < NBLK)
            def _():
                pltpu.sync_copy(
                    acc_sh.at[pl.ds(b * ZB, ZB), :],
                    out_hbm.at[pl.ds((2 * p + c) * N + b * ZB, ZB), :])

            return carry

        lax.fori_loop(0, NBLK_IT, ocp, 0)
        plsc.subcore_barrier()


@functools.cache
def _sc_agg_call():
    return pl.kernel(
        _sc_agg_body,
        out_type=jax.ShapeDtypeStruct((4 * N, DH), jnp.float32),
        mesh=plsc.VectorSubcoreMesh(core_axis_name="c", subcore_axis_name="s"),
        compiler_params=pltpu.CompilerParams(use_tc_tiling_on_sc=False),
        scratch_types=[
            pltpu.VMEM((CHUNK,), jnp.int32),
            pltpu.VMEM((CHUNK,), jnp.int32),
            pltpu.VMEM((CHUNK, DH), jnp.float32),
            pltpu.VMEM((CHUNK, DH), jnp.float32),
            pltpu.VMEM((ZB, DH), jnp.float32),
            pltpu.VMEM_SHARED((N, DH), jnp.float32),
            pltpu.SemaphoreType.DMA,
        ],
    )


def _sc_agg(xa, xb, ea, eb, src, dst):
    return _sc_agg_call()(xa, xb, ea, eb, src, dst)


def _edge_lin_body(ea_ref, w_ref, b_ref, o_ref):
    o_ref[...] = (
        jnp.dot(ea_ref[...], w_ref[...], preferred_element_type=jnp.float32)
        + b_ref[...]
    )


def _edge_lin(edge_attr, w, b):
    return pl.pallas_call(
        _edge_lin_body,
        grid=(E // EB,),
        in_specs=[
            pl.BlockSpec((EB, 16), lambda i: (i, 0)),
            pl.BlockSpec((16, DH), lambda i: (0, 0)),
            pl.BlockSpec((1, DH), lambda i: (0, 0)),
        ],
        out_specs=pl.BlockSpec((EB, DH), lambda i: (i, 0)),
        out_shape=jax.ShapeDtypeStruct((E, DH), jnp.float32),
    )(edge_attr, w, b.reshape(1, DH))


def _mlp_body(xa_ref, xb_ref, pa_ref, pb_ref, w1a_ref, w1b_ref, b1_ref,
              w2_ref, b2_ref, eps_ref, o_ref):
    scale = 1.0 + eps_ref[0, 0]
    ha = xa_ref[...] * scale + pa_ref[0] + pa_ref[1]
    hb = xb_ref[...] * scale + pb_ref[0] + pb_ref[1]
    h = (
        jnp.dot(ha, w1a_ref[...], preferred_element_type=jnp.float32)
        + jnp.dot(hb, w1b_ref[...], preferred_element_type=jnp.float32)
        + b1_ref[...]
    )
    h = jnp.maximum(h, 0.0)
    h = jnp.maximum(
        jnp.dot(h, w2_ref[...], preferred_element_type=jnp.float32)
        + b2_ref[...], 0.0)
    o_ref[...] = h


def _mlp(xa, xb, pa, pb, w1a, w1b, b1, w2, b2, eps):
    return pl.pallas_call(
        _mlp_body,
        grid=(NRB,),
        in_specs=[
            pl.BlockSpec((RB, DH), lambda i: (i, 0)),
            pl.BlockSpec((RB, DH), lambda i: (i, 0)),
            pl.BlockSpec((2, RB, DH), lambda i: (0, i, 0)),
            pl.BlockSpec((2, RB, DH), lambda i: (0, i, 0)),
            pl.BlockSpec((DH, H), lambda i: (0, 0)),
            pl.BlockSpec((DH, H), lambda i: (0, 0)),
            pl.BlockSpec((1, H), lambda i: (0, 0)),
            pl.BlockSpec((H, H), lambda i: (0, 0)),
            pl.BlockSpec((1, H), lambda i: (0, 0)),
            pl.BlockSpec(memory_space=pltpu.SMEM),
        ],
        out_specs=pl.BlockSpec((RB, H), lambda i: (i, 0)),
        out_shape=jax.ShapeDtypeStruct((N, H), jnp.float32),
    )(xa, xb, pa, pb, w1a, w1b, b1.reshape(1, H), w2, b2.reshape(1, H),
      eps.reshape(1, 1))


def _pool_body(x_ref, b_ref, sum_ref, cnt_ref):
    @pl.when(pl.program_id(0) == 0)
    def _():
        sum_ref[...] = jnp.zeros_like(sum_ref)
        cnt_ref[...] = jnp.zeros_like(cnt_ref)

    bb = b_ref[...].reshape(1, RB)
    gi = lax.broadcasted_iota(jnp.int32, (NG, 1), 0)
    one = (bb == gi).astype(jnp.float32)
    sum_ref[...] += jnp.dot(one, x_ref[...],
                            preferred_element_type=jnp.float32)
    cnt_ref[...] += jnp.broadcast_to(
        jnp.sum(one, axis=1, keepdims=True), (NG, H))


def _pool(x_last, batch3):
    return pl.pallas_call(
        _pool_body,
        grid=(NRB,),
        in_specs=[
            pl.BlockSpec((RB, H), lambda i: (i, 0)),
            pl.BlockSpec((1, 1, RB), lambda i: (i, 0, 0)),
        ],
        out_specs=[
            pl.BlockSpec((NG, H), lambda i: (0, 0)),
            pl.BlockSpec((NG, H), lambda i: (0, 0)),
        ],
        out_shape=[
            jax.ShapeDtypeStruct((NG, H), jnp.float32),
            jax.ShapeDtypeStruct((NG, H), jnp.float32),
        ],
    )(x_last, batch3)


def _final_body(s_ref, c_ref, w_ref, b_ref, o_ref):
    pooled = s_ref[...] / jnp.maximum(c_ref[...], 1.0)
    o_ref[...] = (
        jnp.dot(pooled, w_ref[...], preferred_element_type=jnp.float32)
        + b_ref[...]
    )


def _final(sums, cnts, fin_w, fin_b):
    return pl.pallas_call(
        _final_body,
        out_shape=jax.ShapeDtypeStruct((NG, NCLS), jnp.float32),
    )(sums, cnts, fin_w, fin_b.reshape(1, NCLS))


def kernel(x, edge_index, edge_attr, batch, random_feats, params):
    src = edge_index[0]
    dst = edge_index[1]
    padw = DH - (DIN - DH)
    x_l = x
    for lp in params["layers"]:
        xa = x_l[:, :DH]
        xb = jnp.concatenate([x_l[:, DH:], random_feats,
                              jnp.zeros((N, padw), jnp.float32)], axis=1)
        wa, ba = lp["lin_w"][:, :DH], lp["lin_b"][:DH]
        wb = jnp.pad(lp["lin_w"][:, DH:], ((0, 0), (0, padw)))
        bb = jnp.pad(lp["lin_b"][DH:], (0, padw))
        ea = _edge_lin(edge_attr, wa, ba)
        eb = _edge_lin(edge_attr, wb, bb)
        parts = _sc_agg(xa, xb, ea, eb, src, dst).reshape(2, 2, N, DH)
        w1a = lp["w1"][:DH]
        w1b = jnp.pad(lp["w1"][DH:], ((0, padw), (0, 0)))
        x_l = _mlp(xa, xb, parts[0], parts[1], w1a, w1b, lp["b1"],
                   lp["w2"], lp["b2"], lp["eps"])
    sums, cnts = _pool(x_l, batch.reshape(NRB, 1, RB))
    return _final(sums, cnts, params["fin_w"], params["fin_b"])


# speedup vs baseline: 2.5676x; 1.5660x over previous
import functools

import jax
import jax.numpy as jnp
from jax import lax
from jax.experimental import pallas as pl
from jax.experimental.pallas import tpu as pltpu
from jax.experimental.pallas import tpu_sc as plsc

N = 10000
E = 320000
DIN = 144
DH = 80
H = 128
NG = 64
NCLS = 10
LANES = 16

NC = 2
NS = 16
NW = NC * NS
EPW = E // NW
CHUNK = 80
NCHUNK = EPW // CHUNK
ZB = 200
NBLK = N // ZB
NBLK_IT = -(-NBLK // NS)

RB = 2000
NRB = N // RB
EB = 8000


def _sc_agg_body(xa_hbm, xb_hbm, ea_hbm, eb_hbm, src_hbm, dst_hbm, out_hbm,
                 src_all, dst_v, x_v, e_v, z_v, acc_sh, sems):
    c = lax.axis_index("c")
    s = lax.axis_index("s")
    wid = c * NS + s
    ebase = wid * EPW

    pltpu.sync_copy(src_hbm.at[pl.ds(ebase, EPW)], src_all)

    zv = jnp.zeros((LANES,), jnp.float32)

    def zrow(r, carry):
        for j in range(DH // LANES):
            z_v[r, pl.ds(j * LANES, LANES)] = zv
        return carry

    lax.fori_loop(0, ZB, zrow, 0)

    for p, (x_hbm, e_hbm) in enumerate(((xa_hbm, ea_hbm), (xb_hbm, eb_hbm))):
        def zcp(k, carry):
            b = s + NS * k

            @pl.when(b < NBLK)
            def _():
                pltpu.sync_copy(z_v, acc_sh.at[pl.ds(b * ZB, ZB), :])

            return carry

        lax.fori_loop(0, NBLK_IT, zcp, 0)
        plsc.subcore_barrier()

        def issue(i, b):
            pltpu.async_copy(x_hbm.at[src_all.at[pl.ds(i * CHUNK, CHUNK)]],
                             x_v.at[b], sems.at[b, 0])
            pltpu.async_copy(e_hbm.at[pl.ds(ebase + i * CHUNK, CHUNK), :],
                             e_v.at[b], sems.at[b, 1])
            pltpu.async_copy(dst_hbm.at[pl.ds(ebase + i * CHUNK, CHUNK)],
                             dst_v.at[b], sems.at[b, 2])

        def wait(i, b):
            pltpu.make_async_copy(
                x_hbm.at[src_all.at[pl.ds(i * CHUNK, CHUNK)]],
                x_v.at[b], sems.at[b, 0]).wait()
            pltpu.make_async_copy(
                e_hbm.at[pl.ds(ebase + i * CHUNK, CHUNK), :],
                e_v.at[b], sems.at[b, 1]).wait()
            pltpu.make_async_copy(
                dst_hbm.at[pl.ds(ebase + i * CHUNK, CHUNK)],
                dst_v.at[b], sems.at[b, 2]).wait()

        def crunch(b):
            def row(r, inner):
                for j in range(DH // LANES):
                    sl = pl.ds(j * LANES, LANES)
                    x_v[b, r, sl] = jnp.maximum(
                        x_v[b, r, sl] + e_v[b, r, sl], 0.0)
                return inner

            lax.fori_loop(0, CHUNK, row, 0)
            pltpu.sync_copy(x_v.at[b], acc_sh.at[dst_v.at[b]], add=True)

        issue(0, 0)

        def pair_body(t, carry):
            i0 = 2 * t
            issue(i0 + 1, 1)
            wait(i0, 0)
            crunch(0)

            @pl.when(i0 + 2 < NCHUNK)
            def _():
                issue(i0 + 2, 0)

            wait(i0 + 1, 1)
            crunch(1)
            return carry

        lax.fori_loop(0, NCHUNK // 2, pair_body, 0)
        if NCHUNK % 2:
            wait(NCHUNK - 1, 0)
            crunch(0)
        plsc.subcore_barrier()

        def ocp(k, carry):
            b = s + NS * k

            @pl.when(b < NBLK)
            def _():
                pltpu.sync_copy(
                    acc_sh.at[pl.ds(b * ZB, ZB), :],
                    out_hbm.at[pl.ds((2 * p + c) * N + b * ZB, ZB), :])

            return carry

        lax.fori_loop(0, NBLK_IT, ocp, 0)
        plsc.subcore_barrier()


@functools.cache
def _sc_agg_call():
    return pl.kernel(
        _sc_agg_body,
        out_type=jax.ShapeDtypeStruct((4 * N, DH), jnp.float32),
        mesh=plsc.VectorSubcoreMesh(core_axis_name="c", subcore_axis_name="s"),
        compiler_params=pltpu.CompilerParams(use_tc_tiling_on_sc=False),
        scratch_types=[
            pltpu.VMEM((EPW,), jnp.int32),
            pltpu.VMEM((2, CHUNK), jnp.int32),
            pltpu.VMEM((2, CHUNK, DH), jnp.float32),
            pltpu.VMEM((2, CHUNK, DH), jnp.float32),
            pltpu.VMEM((ZB, DH), jnp.float32),
            pltpu.VMEM_SHARED((N, DH), jnp.float32),
            pltpu.SemaphoreType.DMA((2, 3)),
        ],
    )


def _sc_agg(xa, xb, ea, eb, src, dst):
    return _sc_agg_call()(xa, xb, ea, eb, src, dst)


def _edge_lin_body(ea_ref, w_ref, b_ref, o_ref):
    o_ref[...] = (
        jnp.dot(ea_ref[...], w_ref[...], preferred_element_type=jnp.float32)
        + b_ref[...]
    )


def _edge_lin(edge_attr, w, b):
    return pl.pallas_call(
        _edge_lin_body,
        grid=(E // EB,),
        in_specs=[
            pl.BlockSpec((EB, 16), lambda i: (i, 0)),
            pl.BlockSpec((16, DH), lambda i: (0, 0)),
            pl.BlockSpec((1, DH), lambda i: (0, 0)),
        ],
        out_specs=pl.BlockSpec((EB, DH), lambda i: (i, 0)),
        out_shape=jax.ShapeDtypeStruct((E, DH), jnp.float32),
    )(edge_attr, w, b.reshape(1, DH))


def _mlp_body(xa_ref, xb_ref, pa_ref, pb_ref, w1a_ref, w1b_ref, b1_ref,
              w2_ref, b2_ref, eps_ref, o_ref):
    scale = 1.0 + eps_ref[0, 0]
    ha = xa_ref[...] * scale + pa_ref[0] + pa_ref[1]
    hb = xb_ref[...] * scale + pb_ref[0] + pb_ref[1]
    h = (
        jnp.dot(ha, w1a_ref[...], preferred_element_type=jnp.float32)
        + jnp.dot(hb, w1b_ref[...], preferred_element_type=jnp.float32)
        + b1_ref[...]
    )
    h = jnp.maximum(h, 0.0)
    h = jnp.maximum(
        jnp.dot(h, w2_ref[...], preferred_element_type=jnp.float32)
        + b2_ref[...], 0.0)
    o_ref[...] = h


def _mlp(xa, xb, pa, pb, w1a, w1b, b1, w2, b2, eps):
    return pl.pallas_call(
        _mlp_body,
        grid=(NRB,),
        in_specs=[
            pl.BlockSpec((RB, DH), lambda i: (i, 0)),
            pl.BlockSpec((RB, DH), lambda i: (i, 0)),
            pl.BlockSpec((2, RB, DH), lambda i: (0, i, 0)),
            pl.BlockSpec((2, RB, DH), lambda i: (0, i, 0)),
            pl.BlockSpec((DH, H), lambda i: (0, 0)),
            pl.BlockSpec((DH, H), lambda i: (0, 0)),
            pl.BlockSpec((1, H), lambda i: (0, 0)),
            pl.BlockSpec((H, H), lambda i: (0, 0)),
            pl.BlockSpec((1, H), lambda i: (0, 0)),
            pl.BlockSpec(memory_space=pltpu.SMEM),
        ],
        out_specs=pl.BlockSpec((RB, H), lambda i: (i, 0)),
        out_shape=jax.ShapeDtypeStruct((N, H), jnp.float32),
    )(xa, xb, pa, pb, w1a, w1b, b1.reshape(1, H), w2, b2.reshape(1, H),
      eps.reshape(1, 1))


def _pool_body(x_ref, b_ref, sum_ref, cnt_ref):
    @pl.when(pl.program_id(0) == 0)
    def _():
        sum_ref[...] = jnp.zeros_like(sum_ref)
        cnt_ref[...] = jnp.zeros_like(cnt_ref)

    bb = b_ref[...].reshape(1, RB)
    gi = lax.broadcasted_iota(jnp.int32, (NG, 1), 0)
    one = (bb == gi).astype(jnp.float32)
    sum_ref[...] += jnp.dot(one, x_ref[...],
                            preferred_element_type=jnp.float32)
    cnt_ref[...] += jnp.broadcast_to(
        jnp.sum(one, axis=1, keepdims=True), (NG, H))


def _pool(x_last, batch3):
    return pl.pallas_call(
        _pool_body,
        grid=(NRB,),
        in_specs=[
            pl.BlockSpec((RB, H), lambda i: (i, 0)),
            pl.BlockSpec((1, 1, RB), lambda i: (i, 0, 0)),
        ],
        out_specs=[
            pl.BlockSpec((NG, H), lambda i: (0, 0)),
            pl.BlockSpec((NG, H), lambda i: (0, 0)),
        ],
        out_shape=[
            jax.ShapeDtypeStruct((NG, H), jnp.float32),
            jax.ShapeDtypeStruct((NG, H), jnp.float32),
        ],
    )(x_last, batch3)


def _final_body(s_ref, c_ref, w_ref, b_ref, o_ref):
    pooled = s_ref[...] / jnp.maximum(c_ref[...], 1.0)
    o_ref[...] = (
        jnp.dot(pooled, w_ref[...], preferred_element_type=jnp.float32)
        + b_ref[...]
    )


def _final(sums, cnts, fin_w, fin_b):
    return pl.pallas_call(
        _final_body,
        out_shape=jax.ShapeDtypeStruct((NG, NCLS), jnp.float32),
    )(sums, cnts, fin_w, fin_b.reshape(1, NCLS))


def kernel(x, edge_index, edge_attr, batch, random_feats, params):
    src = edge_index[0]
    dst = edge_index[1]
    padw = DH - (DIN - DH)
    x_l = x
    for lp in params["layers"]:
        xa = x_l[:, :DH]
        xb = jnp.concatenate([x_l[:, DH:], random_feats,
                              jnp.zeros((N, padw), jnp.float32)], axis=1)
        wa, ba = lp["lin_w"][:, :DH], lp["lin_b"][:DH]
        wb = jnp.pad(lp["lin_w"][:, DH:], ((0, 0), (0, padw)))
        bb = jnp.pad(lp["lin_b"][DH:], (0, padw))
        ea = _edge_lin(edge_attr, wa, ba)
        eb = _edge_lin(edge_attr, wb, bb)
        parts = _sc_agg(xa, xb, ea, eb, src, dst).reshape(2, 2, N, DH)
        w1a = lp["w1"][:DH]
        w1b = jnp.pad(lp["w1"][DH:], ((0, padw), (0, 0)))
        x_l = _mlp(xa, xb, parts[0], parts[1], w1a, w1b, lp["b1"],
                   lp["w2"], lp["b2"], lp["eps"])
    sums, cnts = _pool(x_l, batch.reshape(NRB, 1, RB))
    return _final(sums, cnts, params["fin_w"], params["fin_b"])


# speedup vs baseline: 4.3513x; 1.6947x over previous
import functools

import jax
import jax.numpy as jnp
from jax import lax
from jax.experimental import pallas as pl
from jax.experimental.pallas import tpu as pltpu
from jax.experimental.pallas import tpu_sc as plsc

N = 10000
E = 320000
DA = 128
PW = 64
DB = 16
NL = 3
DBL = DB * NL
H = 128
NG = 64
NCLS = 10
LANES = 16

NC = 2
NS = 16
NW = NC * NS
EPW = E // NW
CHUNK = 80
NCHUNK = EPW // CHUNK
ZB = 200
NBLK = N // ZB
NBLK_IT = -(-NBLK // NS)

RB = 2000
NRB = N // RB
EB = 8000


def _sc_a_body(x_hbm, e_hbm, src_hbm, dst_hbm, out_hbm,
               src_all, xidx, eidx, dst_v, x_v, e_v, z_v, acc_sh, sems):
    c = lax.axis_index("c")
    s = lax.axis_index("s")
    wid = c * NS + s
    ebase = wid * EPW

    pltpu.sync_copy(src_hbm.at[pl.ds(ebase, EPW)], src_all)

    zv = jnp.zeros((LANES,), jnp.float32)

    def zrow(r, carry):
        for j in range(PW // LANES):
            z_v[r, pl.ds(j * LANES, LANES)] = zv
        return carry

    lax.fori_loop(0, ZB, zrow, 0)

    iot = lax.iota(jnp.int32, LANES)

    for p in range(DA // PW):
        def xrow(k, carry):
            sl = pl.ds(k * LANES, LANES)
            xidx[sl] = src_all[sl] * 2 + p
            return carry

        lax.fori_loop(0, EPW // LANES, xrow, 0)

        def erow(k, carry):
            sl = pl.ds(k * LANES, LANES)
            eidx[sl] = (2 * ebase + p + 32 * k) + iot * 2
            return carry

        lax.fori_loop(0, EPW // LANES, erow, 0)

        def zcp(k, carry):
            b = s + NS * k

            @pl.when(b < NBLK)
            def _():
                pltpu.sync_copy(z_v, acc_sh.at[pl.ds(b * ZB, ZB), :])

            return carry

        lax.fori_loop(0, NBLK_IT, zcp, 0)
        plsc.subcore_barrier()

        def issue(i, b):
            pltpu.async_copy(x_hbm.at[xidx.at[pl.ds(i * CHUNK, CHUNK)]],
                             x_v.at[b], sems.at[b, 0])
            pltpu.async_copy(e_hbm.at[eidx.at[pl.ds(i * CHUNK, CHUNK)]],
                             e_v.at[b], sems.at[b, 1])
            pltpu.async_copy(dst_hbm.at[pl.ds(ebase + i * CHUNK, CHUNK)],
                             dst_v.at[b], sems.at[b, 2])

        def wait(i, b):
            pltpu.make_async_copy(
                x_hbm.at[xidx.at[pl.ds(i * CHUNK, CHUNK)]],
                x_v.at[b], sems.at[b, 0]).wait()
            pltpu.make_async_copy(
                e_hbm.at[eidx.at[pl.ds(i * CHUNK, CHUNK)]],
                e_v.at[b], sems.at[b, 1]).wait()
            pltpu.make_async_copy(
                dst_hbm.at[pl.ds(ebase + i * CHUNK, CHUNK)],
                dst_v.at[b], sems.at[b, 2]).wait()

        def crunch(b):
            def row(r, inner):
                for j in range(PW // LANES):
                    sl = pl.ds(j * LANES, LANES)
                    x_v[b, r, sl] = jnp.maximum(
                        x_v[b, r, sl] + e_v[b, r, sl], 0.0)
                return inner

            lax.fori_loop(0, CHUNK, row, 0)
            pltpu.sync_copy(x_v.at[b], acc_sh.at[dst_v.at[b]], add=True)

        issue(0, 0)

        def pair_body(t, carry):
            i0 = 2 * t
            issue(i0 + 1, 1)
            wait(i0, 0)
            crunch(0)

            @pl.when(i0 + 2 < NCHUNK)
            def _():
                issue(i0 + 2, 0)

            wait(i0 + 1, 1)
            crunch(1)
            return carry

        lax.fori_loop(0, NCHUNK // 2, pair_body, 0)
        if NCHUNK % 2:
            wait(NCHUNK - 1, 0)
            crunch(0)
        plsc.subcore_barrier()

        def ocp(k, carry):
            b = s + NS * k

            @pl.when(b < NBLK)
            def _():
                pltpu.sync_copy(
                    acc_sh.at[pl.ds(b * ZB, ZB), :],
                    out_hbm.at[pl.ds((2 * p + c) * N + b * ZB, ZB), :])

            return carry

        lax.fori_loop(0, NBLK_IT, ocp, 0)
        plsc.subcore_barrier()


@functools.cache
def _sc_a_call():
    return pl.kernel(
        _sc_a_body,
        out_type=jax.ShapeDtypeStruct((4 * N, PW), jnp.float32),
        mesh=plsc.VectorSubcoreMesh(core_axis_name="c", subcore_axis_name="s"),
        compiler_params=pltpu.CompilerParams(use_tc_tiling_on_sc=False),
        scratch_types=[
            pltpu.VMEM((EPW,), jnp.int32),
            pltpu.VMEM((EPW,), jnp.int32),
            pltpu.VMEM((EPW,), jnp.int32),
            pltpu.VMEM((2, CHUNK), jnp.int32),
            pltpu.VMEM((2, CHUNK, PW), jnp.float32),
            pltpu.VMEM((2, CHUNK, PW), jnp.float32),
            pltpu.VMEM((ZB, PW), jnp.float32),
            pltpu.VMEM_SHARED((N, PW), jnp.float32),
            pltpu.SemaphoreType.DMA((2, 3)),
        ],
    )


def _sc_b_body(rf_hbm, e_hbm, src_hbm, dst_hbm, out_hbm,
               src_all, dst_v, rf_v, e_v, z_v, acc_sh, sems):
    c = lax.axis_index("c")
    s = lax.axis_index("s")
    wid = c * NS + s
    ebase = wid * EPW

    pltpu.sync_copy(src_hbm.at[pl.ds(ebase, EPW)], src_all)

    zv = jnp.zeros((LANES,), jnp.float32)

    def zrow(r, carry):
        for j in range(DBL // LANES):
            z_v[r, pl.ds(j * LANES, LANES)] = zv
        return carry

    lax.fori_loop(0, ZB, zrow, 0)

    def zcp(k, carry):
        b = s + NS * k

        @pl.when(b < NBLK)
        def _():
            pltpu.sync_copy(z_v, acc_sh.at[pl.ds(b * ZB, ZB), :])

        return carry

    lax.fori_loop(0, NBLK_IT, zcp, 0)
    plsc.subcore_barrier()

    def issue(i, b):
        pltpu.async_copy(rf_hbm.at[src_all.at[pl.ds(i * CHUNK, CHUNK)]],
                         rf_v.at[b], sems.at[b, 0])
        pltpu.async_copy(e_hbm.at[pl.ds(ebase + i * CHUNK, CHUNK), :],
                         e_v.at[b], sems.at[b, 1])
        pltpu.async_copy(dst_hbm.at[pl.ds(ebase + i * CHUNK, CHUNK)],
                         dst_v.at[b], sems.at[b, 2])

    def wait(i, b):
        pltpu.make_async_copy(
            rf_hbm.at[src_all.at[pl.ds(i * CHUNK, CHUNK)]],
            rf_v.at[b], sems.at[b, 0]).wait()
        pltpu.make_async_copy(
            e_hbm.at[pl.ds(ebase + i * CHUNK, CHUNK), :],
            e_v.at[b], sems.at[b, 1]).wait()
        pltpu.make_async_copy(
            dst_hbm.at[pl.ds(ebase + i * CHUNK, CHUNK)],
            dst_v.at[b], sems.at[b, 2]).wait()

    def crunch(b):
        def row(r, inner):
            rv = rf_v[b, r, :]
            for j in range(NL):
                sl = pl.ds(j * DB, DB)
                e_v[b, r, sl] = jnp.maximum(e_v[b, r, sl] + rv, 0.0)
            return inner

        lax.fori_loop(0, CHUNK, row, 0)
        pltpu.sync_copy(e_v.at[b], acc_sh.at[dst_v.at[b]], add=True)

    issue(0, 0)

    def pair_body(t, carry):
        i0 = 2 * t
        issue(i0 + 1, 1)
        wait(i0, 0)
        crunch(0)

        @pl.when(i0 + 2 < NCHUNK)
        def _():
            issue(i0 + 2, 0)

        wait(i0 + 1, 1)
        crunch(1)
        return carry

    lax.fori_loop(0, NCHUNK // 2, pair_body, 0)
    if NCHUNK % 2:
        wait(NCHUNK - 1, 0)
        crunch(0)
    plsc.subcore_barrier()

    def ocp(k, carry):
        b = s + NS * k

        @pl.when(b < NBLK)
        def _():
            pltpu.sync_copy(
                acc_sh.at[pl.ds(b * ZB, ZB), :],
                out_hbm.at[pl.ds(c * N + b * ZB, ZB), :])

        return carry

    lax.fori_loop(0, NBLK_IT, ocp, 0)
    plsc.subcore_barrier()


@functools.cache
def _sc_b_call():
    return pl.kernel(
        _sc_b_body,
        out_type=jax.ShapeDtypeStruct((2 * N, DBL), jnp.float32),
        mesh=plsc.VectorSubcoreMesh(core_axis_name="c", subcore_axis_name="s"),
        compiler_params=pltpu.CompilerParams(use_tc_tiling_on_sc=False),
        scratch_types=[
            pltpu.VMEM((EPW,), jnp.int32),
            pltpu.VMEM((2, CHUNK), jnp.int32),
            pltpu.VMEM((2, CHUNK, DB), jnp.float32),
            pltpu.VMEM((2, CHUNK, DBL), jnp.float32),
            pltpu.VMEM((ZB, DBL), jnp.float32),
            pltpu.VMEM_SHARED((N, DBL), jnp.float32),
            pltpu.SemaphoreType.DMA((2, 3)),
        ],
    )


def _edge_lin_body(ea_ref, w_ref, b_ref, o_ref):
    o_ref[...] = (
        jnp.dot(ea_ref[...], w_ref[...], preferred_element_type=jnp.float32)
        + b_ref[...]
    )


def _edge_lin(edge_attr, w, b, d):
    return pl.pallas_call(
        _edge_lin_body,
        grid=(E // EB,),
        in_specs=[
            pl.BlockSpec((EB, 16), lambda i: (i, 0)),
            pl.BlockSpec((16, d), lambda i: (0, 0)),
            pl.BlockSpec((1, d), lambda i: (0, 0)),
        ],
        out_specs=pl.BlockSpec((EB, d), lambda i: (i, 0)),
        out_shape=jax.ShapeDtypeStruct((E, d), jnp.float32),
    )(edge_attr, w, b.reshape(1, d))


def _mlp_body(xa_ref, rf_ref, pa_ref, pb_ref, w1lo_ref, w1hi_ref, w1b_ref,
              b1_ref, w2_ref, b2_ref, eps_ref, o_ref):
    scale = 1.0 + eps_ref[0, 0]
    xa = xa_ref[...]
    hlo = xa[:, :PW] * scale + pa_ref[0, 0] + pa_ref[0, 1]
    hhi = xa[:, PW:] * scale + pa_ref[1, 0] + pa_ref[1, 1]
    hb = rf_ref[...] * scale + pb_ref[0] + pb_ref[1]
    h = (
        jnp.dot(hlo, w1lo_ref[...], preferred_element_type=jnp.float32)
        + jnp.dot(hhi, w1hi_ref[...], preferred_element_type=jnp.float32)
        + jnp.dot(hb, w1b_ref[...], preferred_element_type=jnp.float32)
        + b1_ref[...]
    )
    h = jnp.maximum(h, 0.0)
    h = jnp.maximum(
        jnp.dot(h, w2_ref[...], preferred_element_type=jnp.float32)
        + b2_ref[...], 0.0)
    o_ref[...] = h


def _mlp(xa, rf, pa, pb, w1, w1b, b1, w2, b2, eps):
    return pl.pallas_call(
        _mlp_body,
        grid=(NRB,),
        in_specs=[
            pl.BlockSpec((RB, DA), lambda i: (i, 0)),
            pl.BlockSpec((RB, DB), lambda i: (i, 0)),
            pl.BlockSpec((2, 2, RB, PW), lambda i: (0, 0, i, 0)),
            pl.BlockSpec((2, RB, DB), lambda i: (0, i, 0)),
            pl.BlockSpec((PW, H), lambda i: (0, 0)),
            pl.BlockSpec((PW, H), lambda i: (0, 0)),
            pl.BlockSpec((DB, H), lambda i: (0, 0)),
            pl.BlockSpec((1, H), lambda i: (0, 0)),
            pl.BlockSpec((H, H), lambda i: (0, 0)),
            pl.BlockSpec((1, H), lambda i: (0, 0)),
            pl.BlockSpec(memory_space=pltpu.SMEM),
        ],
        out_specs=pl.BlockSpec((RB, H), lambda i: (i, 0)),
        out_shape=jax.ShapeDtypeStruct((N, H), jnp.float32),
    )(xa, rf, pa, pb, w1[:PW], w1[PW:DA], w1b, b1.reshape(1, H), w2,
      b2.reshape(1, H), eps.reshape(1, 1))


def _pool_body(x_ref, b_ref, sum_ref, cnt_ref):
    @pl.when(pl.program_id(0) == 0)
    def _():
        sum_ref[...] = jnp.zeros_like(sum_ref)
        cnt_ref[...] = jnp.zeros_like(cnt_ref)

    bb = b_ref[...].reshape(1, RB)
    gi = lax.broadcasted_iota(jnp.int32, (NG, 1), 0)
    one = (bb == gi).astype(jnp.float32)
    sum_ref[...] += jnp.dot(one, x_ref[...],
                            preferred_element_type=jnp.float32)
    cnt_ref[...] += jnp.broadcast_to(
        jnp.sum(one, axis=1, keepdims=True), (NG, H))


def _pool(x_last, batch3):
    return pl.pallas_call(
        _pool_body,
        grid=(NRB,),
        in_specs=[
            pl.BlockSpec((RB, H), lambda i: (i, 0)),
            pl.BlockSpec((1, 1, RB), lambda i: (i, 0, 0)),
        ],
        out_specs=[
            pl.BlockSpec((NG, H), lambda i: (0, 0)),
            pl.BlockSpec((NG, H), lambda i: (0, 0)),
        ],
        out_shape=[
            jax.ShapeDtypeStruct((NG, H), jnp.float32),
            jax.ShapeDtypeStruct((NG, H), jnp.float32),
        ],
    )(x_last, batch3)


def _final_body(s_ref, c_ref, w_ref, b_ref, o_ref):
    pooled = s_ref[...] / jnp.maximum(c_ref[...], 1.0)
    o_ref[...] = (
        jnp.dot(pooled, w_ref[...], preferred_element_type=jnp.float32)
        + b_ref[...]
    )


def _final(sums, cnts, fin_w, fin_b):
    return pl.pallas_call(
        _final_body,
        out_shape=jax.ShapeDtypeStruct((NG, NCLS), jnp.float32),
    )(sums, cnts, fin_w, fin_b.reshape(1, NCLS))


def kernel(x, edge_index, edge_attr, batch, random_feats, params):
    src = edge_index[0]
    dst = edge_index[1]
    layers = params["layers"]

    wb_all = jnp.concatenate([lp["lin_w"][:, DA:] for lp in layers], axis=1)
    bb_all = jnp.concatenate([lp["lin_b"][DA:] for lp in layers])
    eb_all = _edge_lin(edge_attr, wb_all, bb_all, DBL)
    parts_b = _sc_b_call()(random_feats, eb_all, src, dst).reshape(2, N, DBL)

    x_l = x
    for li, lp in enumerate(layers):
        ea = _edge_lin(edge_attr, lp["lin_w"][:, :DA], lp["lin_b"][:DA], DA)
        pa = _sc_a_call()(x_l.reshape(2 * N, PW), ea.reshape(2 * E, PW),
                          src, dst).reshape(2, 2, N, PW)
        pb = lax.slice_in_dim(parts_b, li * DB, (li + 1) * DB, axis=2)
        x_l = _mlp(x_l, random_feats, pa, pb, lp["w1"][:DA], lp["w1"][DA:],
                   lp["b1"], lp["w2"], lp["b2"], lp["eps"])
    sums, cnts = _pool(x_l, batch.reshape(NRB, 1, RB))
    return _final(sums, cnts, params["fin_w"], params["fin_b"])


# speedup vs baseline: 4.4837x; 1.0304x over previous
import functools

import jax
import jax.numpy as jnp
from jax import lax
from jax.experimental import pallas as pl
from jax.experimental.pallas import tpu as pltpu
from jax.experimental.pallas import tpu_sc as plsc

N = 10000
E = 320000
DA = 128
PW = 64
DB = 16
NL = 3
DBL = DB * NL
H = 128
NG = 64
NCLS = 10
LANES = 16

NC = 2
NS = 16
NW = NC * NS
EPW = E // NW
CHUNK = 80
NCHUNK = EPW // CHUNK
ZB = 200
NBLK = N // ZB
NBLK_IT = -(-NBLK // NS)

RB = 2000
NRB = N // RB
EB = 8000


def _sc_a_body(x_hbm, e_hbm, src_hbm, dst_hbm, out_hbm,
               src_all, xidx, eidx, dst_v, x_v, e_v, z_v, acc_sh, sems):
    c = lax.axis_index("c")
    s = lax.axis_index("s")
    wid = c * NS + s
    ebase = wid * EPW

    pltpu.sync_copy(src_hbm.at[pl.ds(ebase, EPW)], src_all)

    zv = jnp.zeros((LANES,), jnp.float32)

    def zrow(r, carry):
        for j in range(PW // LANES):
            z_v[r, pl.ds(j * LANES, LANES)] = zv
        return carry

    lax.fori_loop(0, ZB, zrow, 0)

    iot = lax.iota(jnp.int32, LANES)

    for p in range(DA // PW):
        def xrow(k, carry):
            sl = pl.ds(k * LANES, LANES)
            xidx[sl] = src_all[sl] * 2 + p
            return carry

        lax.fori_loop(0, EPW // LANES, xrow, 0)

        def erow(k, carry):
            sl = pl.ds(k * LANES, LANES)
            eidx[sl] = (2 * ebase + p + 32 * k) + iot * 2
            return carry

        lax.fori_loop(0, EPW // LANES, erow, 0)

        def zcp(k, carry):
            b = s + NS * k

            @pl.when(b < NBLK)
            def _():
                pltpu.sync_copy(z_v, acc_sh.at[pl.ds(b * ZB, ZB), :])

            return carry

        lax.fori_loop(0, NBLK_IT, zcp, 0)
        plsc.subcore_barrier()

        def issue(i, b):
            pltpu.async_copy(x_hbm.at[xidx.at[pl.ds(i * CHUNK, CHUNK)]],
                             x_v.at[b], sems.at[b, 0])
            pltpu.async_copy(e_hbm.at[eidx.at[pl.ds(i * CHUNK, CHUNK)]],
                             e_v.at[b], sems.at[b, 1])
            pltpu.async_copy(dst_hbm.at[pl.ds(ebase + i * CHUNK, CHUNK)],
                             dst_v.at[b], sems.at[b, 2])

        def wait(i, b):
            pltpu.make_async_copy(
                x_hbm.at[xidx.at[pl.ds(i * CHUNK, CHUNK)]],
                x_v.at[b], sems.at[b, 0]).wait()
            pltpu.make_async_copy(
                e_hbm.at[eidx.at[pl.ds(i * CHUNK, CHUNK)]],
                e_v.at[b], sems.at[b, 1]).wait()
            pltpu.make_async_copy(
                dst_hbm.at[pl.ds(ebase + i * CHUNK, CHUNK)],
                dst_v.at[b], sems.at[b, 2]).wait()

        def crunch(b):
            def row(r, inner):
                for j in range(PW // LANES):
                    sl = pl.ds(j * LANES, LANES)
                    x_v[b, r, sl] = jnp.maximum(
                        x_v[b, r, sl] + e_v[b, r, sl], 0.0)
                return inner

            lax.fori_loop(0, CHUNK, row, 0)
            pltpu.sync_copy(x_v.at[b], acc_sh.at[dst_v.at[b]], add=True)

        issue(0, 0)

        def pair_body(t, carry):
            i0 = 2 * t
            issue(i0 + 1, 1)
            wait(i0, 0)
            crunch(0)

            @pl.when(i0 + 2 < NCHUNK)
            def _():
                issue(i0 + 2, 0)

            wait(i0 + 1, 1)
            crunch(1)
            return carry

        lax.fori_loop(0, NCHUNK // 2, pair_body, 0)
        if NCHUNK % 2:
            wait(NCHUNK - 1, 0)
            crunch(0)
        plsc.subcore_barrier()

        def ocp(k, carry):
            b = s + NS * k

            @pl.when(b < NBLK)
            def _():
                pltpu.sync_copy(
                    acc_sh.at[pl.ds(b * ZB, ZB), :],
                    out_hbm.at[pl.ds((2 * p + c) * N + b * ZB, ZB), :])

            return carry

        lax.fori_loop(0, NBLK_IT, ocp, 0)
        plsc.subcore_barrier()


@functools.cache
def _sc_a_call():
    return pl.kernel(
        _sc_a_body,
        out_type=jax.ShapeDtypeStruct((4 * N, PW), jnp.float32),
        mesh=plsc.VectorSubcoreMesh(core_axis_name="c", subcore_axis_name="s"),
        compiler_params=pltpu.CompilerParams(use_tc_tiling_on_sc=False),
        scratch_types=[
            pltpu.VMEM((EPW,), jnp.int32),
            pltpu.VMEM((EPW,), jnp.int32),
            pltpu.VMEM((EPW,), jnp.int32),
            pltpu.VMEM((2, CHUNK), jnp.int32),
            pltpu.VMEM((2, CHUNK, PW), jnp.float32),
            pltpu.VMEM((2, CHUNK, PW), jnp.float32),
            pltpu.VMEM((ZB, PW), jnp.float32),
            pltpu.VMEM_SHARED((N, PW), jnp.float32),
            pltpu.SemaphoreType.DMA((2, 3)),
        ],
    )


def _sc_b_body(rf_hbm, e_hbm, src_hbm, dst_hbm, out_hbm,
               src_all, eidx, dst_v, rf_v, e_v, m_v, z_v, acc_sh, sems):
    c = lax.axis_index("c")
    s = lax.axis_index("s")
    wid = c * NS + s
    ebase = wid * EPW

    pltpu.sync_copy(src_hbm.at[pl.ds(ebase, EPW)], src_all)

    iot = lax.iota(jnp.int32, LANES)

    def erow(k, carry):
        sl = pl.ds(k * LANES, LANES)
        eidx[sl] = (2 * ebase + 32 * k) + iot * 2
        return carry

    lax.fori_loop(0, EPW // LANES, erow, 0)

    zv = jnp.zeros((LANES,), jnp.float32)

    def zrow(r, carry):
        for j in range(DBL // LANES):
            z_v[r, pl.ds(j * LANES, LANES)] = zv
        return carry

    lax.fori_loop(0, ZB, zrow, 0)

    def zcp(k, carry):
        b = s + NS * k

        @pl.when(b < NBLK)
        def _():
            pltpu.sync_copy(z_v, acc_sh.at[pl.ds(b * ZB, ZB), :])

        return carry

    lax.fori_loop(0, NBLK_IT, zcp, 0)
    plsc.subcore_barrier()

    def issue(i, b):
        pltpu.async_copy(rf_hbm.at[src_all.at[pl.ds(i * CHUNK, CHUNK)]],
                         rf_v.at[b], sems.at[b, 0])
        pltpu.async_copy(e_hbm.at[eidx.at[pl.ds(i * CHUNK, CHUNK)]],
                         e_v.at[b], sems.at[b, 1])
        pltpu.async_copy(dst_hbm.at[pl.ds(ebase + i * CHUNK, CHUNK)],
                         dst_v.at[b], sems.at[b, 2])

    def wait(i, b):
        pltpu.make_async_copy(
            rf_hbm.at[src_all.at[pl.ds(i * CHUNK, CHUNK)]],
            rf_v.at[b], sems.at[b, 0]).wait()
        pltpu.make_async_copy(
            e_hbm.at[eidx.at[pl.ds(i * CHUNK, CHUNK)]],
            e_v.at[b], sems.at[b, 1]).wait()
        pltpu.make_async_copy(
            dst_hbm.at[pl.ds(ebase + i * CHUNK, CHUNK)],
            dst_v.at[b], sems.at[b, 2]).wait()

    def crunch(b):
        def row(r, inner):
            rv = rf_v[b, r, :]
            for j in range(NL):
                sl = pl.ds(j * DB, DB)
                m_v[b, r, sl] = jnp.maximum(e_v[b, r, sl] + rv, 0.0)
            return inner

        lax.fori_loop(0, CHUNK, row, 0)
        pltpu.sync_copy(m_v.at[b], acc_sh.at[dst_v.at[b]], add=True)

    issue(0, 0)

    def pair_body(t, carry):
        i0 = 2 * t
        issue(i0 + 1, 1)
        wait(i0, 0)
        crunch(0)

        @pl.when(i0 + 2 < NCHUNK)
        def _():
            issue(i0 + 2, 0)

        wait(i0 + 1, 1)
        crunch(1)
        return carry

    lax.fori_loop(0, NCHUNK // 2, pair_body, 0)
    if NCHUNK % 2:
        wait(NCHUNK - 1, 0)
        crunch(0)
    plsc.subcore_barrier()

    def ocp(k, carry):
        b = s + NS * k

        @pl.when(b < NBLK)
        def _():
            pltpu.sync_copy(
                acc_sh.at[pl.ds(b * ZB, ZB), :],
                out_hbm.at[pl.ds(c * N + b * ZB, ZB), :])

        return carry

    lax.fori_loop(0, NBLK_IT, ocp, 0)
    plsc.subcore_barrier()


@functools.cache
def _sc_b_call():
    return pl.kernel(
        _sc_b_body,
        out_type=jax.ShapeDtypeStruct((2 * N, DBL), jnp.float32),
        mesh=plsc.VectorSubcoreMesh(core_axis_name="c", subcore_axis_name="s"),
        compiler_params=pltpu.CompilerParams(use_tc_tiling_on_sc=False),
        scratch_types=[
            pltpu.VMEM((EPW,), jnp.int32),
            pltpu.VMEM((EPW,), jnp.int32),
            pltpu.VMEM((2, CHUNK), jnp.int32),
            pltpu.VMEM((2, CHUNK, DB), jnp.float32),
            pltpu.VMEM((2, CHUNK, PW), jnp.float32),
            pltpu.VMEM((2, CHUNK, DBL), jnp.float32),
            pltpu.VMEM((ZB, DBL), jnp.float32),
            pltpu.VMEM_SHARED((N, DBL), jnp.float32),
            pltpu.SemaphoreType.DMA((2, 3)),
        ],
    )


def _edge_lin_body(ea_ref, w_ref, b_ref, o_ref):
    o_ref[...] = (
        jnp.dot(ea_ref[...], w_ref[...], preferred_element_type=jnp.float32)
        + b_ref[...]
    )


def _edge_lin(edge_attr, w, b, d):
    return pl.pallas_call(
        _edge_lin_body,
        grid=(E // EB,),
        in_specs=[
            pl.BlockSpec((EB, 16), lambda i: (i, 0)),
            pl.BlockSpec((16, d), lambda i: (0, 0)),
            pl.BlockSpec((1, d), lambda i: (0, 0)),
        ],
        out_specs=pl.BlockSpec((EB, d), lambda i: (i, 0)),
        out_shape=jax.ShapeDtypeStruct((E, d), jnp.float32),
    )(edge_attr, w, b.reshape(1, d))


def _mlp_body(xa_ref, rf_ref, pa_ref, pb_ref, w1lo_ref, w1hi_ref, w1b_ref,
              b1_ref, w2_ref, b2_ref, eps_ref, o_ref):
    scale = 1.0 + eps_ref[0, 0]
    xa = xa_ref[...]
    hlo = xa[:, :PW] * scale + pa_ref[0, 0] + pa_ref[0, 1]
    hhi = xa[:, PW:] * scale + pa_ref[1, 0] + pa_ref[1, 1]
    hb = rf_ref[...] * scale + pb_ref[0] + pb_ref[1]
    h = (
        jnp.dot(hlo, w1lo_ref[...], preferred_element_type=jnp.float32)
        + jnp.dot(hhi, w1hi_ref[...], preferred_element_type=jnp.float32)
        + jnp.dot(hb, w1b_ref[...], preferred_element_type=jnp.float32)
        + b1_ref[...]
    )
    h = jnp.maximum(h, 0.0)
    h = jnp.maximum(
        jnp.dot(h, w2_ref[...], preferred_element_type=jnp.float32)
        + b2_ref[...], 0.0)
    o_ref[...] = h


def _mlp(xa, rf, pa, pb, w1, w1b, b1, w2, b2, eps):
    return pl.pallas_call(
        _mlp_body,
        grid=(NRB,),
        in_specs=[
            pl.BlockSpec((RB, DA), lambda i: (i, 0)),
            pl.BlockSpec((RB, DB), lambda i: (i, 0)),
            pl.BlockSpec((2, 2, RB, PW), lambda i: (0, 0, i, 0)),
            pl.BlockSpec((2, RB, DB), lambda i: (0, i, 0)),
            pl.BlockSpec((PW, H), lambda i: (0, 0)),
            pl.BlockSpec((PW, H), lambda i: (0, 0)),
            pl.BlockSpec((DB, H), lambda i: (0, 0)),
            pl.BlockSpec((1, H), lambda i: (0, 0)),
            pl.BlockSpec((H, H), lambda i: (0, 0)),
            pl.BlockSpec((1, H), lambda i: (0, 0)),
            pl.BlockSpec(memory_space=pltpu.SMEM),
        ],
        out_specs=pl.BlockSpec((RB, H), lambda i: (i, 0)),
        out_shape=jax.ShapeDtypeStruct((N, H), jnp.float32),
    )(xa, rf, pa, pb, w1[:PW], w1[PW:DA], w1b, b1.reshape(1, H), w2,
      b2.reshape(1, H), eps.reshape(1, 1))


def _pool_body(x_ref, b_ref, sum_ref, cnt_ref):
    @pl.when(pl.program_id(0) == 0)
    def _():
        sum_ref[...] = jnp.zeros_like(sum_ref)
        cnt_ref[...] = jnp.zeros_like(cnt_ref)

    bb = b_ref[...].reshape(1, RB)
    gi = lax.broadcasted_iota(jnp.int32, (NG, 1), 0)
    one = (bb == gi).astype(jnp.float32)
    sum_ref[...] += jnp.dot(one, x_ref[...],
                            preferred_element_type=jnp.float32)
    cnt_ref[...] += jnp.broadcast_to(
        jnp.sum(one, axis=1, keepdims=True), (NG, H))


def _pool(x_last, batch3):
    return pl.pallas_call(
        _pool_body,
        grid=(NRB,),
        in_specs=[
            pl.BlockSpec((RB, H), lambda i: (i, 0)),
            pl.BlockSpec((1, 1, RB), lambda i: (i, 0, 0)),
        ],
        out_specs=[
            pl.BlockSpec((NG, H), lambda i: (0, 0)),
            pl.BlockSpec((NG, H), lambda i: (0, 0)),
        ],
        out_shape=[
            jax.ShapeDtypeStruct((NG, H), jnp.float32),
            jax.ShapeDtypeStruct((NG, H), jnp.float32),
        ],
    )(x_last, batch3)


def _final_body(s_ref, c_ref, w_ref, b_ref, o_ref):
    pooled = s_ref[...] / jnp.maximum(c_ref[...], 1.0)
    o_ref[...] = (
        jnp.dot(pooled, w_ref[...], preferred_element_type=jnp.float32)
        + b_ref[...]
    )


def _final(sums, cnts, fin_w, fin_b):
    return pl.pallas_call(
        _final_body,
        out_shape=jax.ShapeDtypeStruct((NG, NCLS), jnp.float32),
    )(sums, cnts, fin_w, fin_b.reshape(1, NCLS))


def kernel(x, edge_index, edge_attr, batch, random_feats, params):
    src = edge_index[0]
    dst = edge_index[1]
    layers = params["layers"]

    wb_all = jnp.concatenate(
        [lp["lin_w"][:, DA:] for lp in layers]
        + [jnp.zeros((16, DA - DBL), jnp.float32)], axis=1)
    bb_all = jnp.concatenate(
        [lp["lin_b"][DA:] for lp in layers]
        + [jnp.zeros((DA - DBL,), jnp.float32)])
    eb_all = _edge_lin(edge_attr, wb_all, bb_all, DA)
    parts_b = _sc_b_call()(random_feats, eb_all.reshape(2 * E, PW),
                           src, dst).reshape(2, N, DBL)

    x_l = x
    for li, lp in enumerate(layers):
        ea = _edge_lin(edge_attr, lp["lin_w"][:, :DA], lp["lin_b"][:DA], DA)
        pa = _sc_a_call()(x_l.reshape(2 * N, PW), ea.reshape(2 * E, PW),
                          src, dst).reshape(2, 2, N, PW)
        pb = lax.slice_in_dim(parts_b, li * DB, (li + 1) * DB, axis=2)
        x_l = _mlp(x_l, random_feats, pa, pb, lp["w1"][:DA], lp["w1"][DA:],
                   lp["b1"], lp["w2"], lp["b2"], lp["eps"])
    sums, cnts = _pool(x_l, batch.reshape(NRB, 1, RB))
    return _final(sums, cnts, params["fin_w"], params["fin_b"])
